# baseline (speedup 1.0000x reference)
import functools

import jax
import jax.numpy as jnp
from jax import lax
from jax.experimental import pallas as pl
from jax.experimental.pallas import tpu as pltpu
from jax.experimental.pallas import tpu_sc as plsc

_NBUF = 11


def _gather_call(B, D, V):
    info = plsc.get_sparse_core_info()
    NC, NS = info.num_cores, info.num_subcores
    NW = NC * NS
    b_per_w = B // NW
    n_batches = b_per_w // _NBUF
    n_rem = b_per_w % _NBUF

    mesh = plsc.VectorSubcoreMesh(core_axis_name="c", subcore_axis_name="s")

    @functools.partial(
        pl.kernel,
        mesh=mesh,
        out_type=jax.ShapeDtypeStruct((B, 1, D), jnp.float32),
        compiler_params=pltpu.CompilerParams(
            needs_layout_passes=False, disable_bounds_checks=True
        ),
        scratch_types=[
            pltpu.VMEM((b_per_w + 16,), jnp.int32),
            pltpu.VMEM((_NBUF, D, 128), jnp.float32),
            pltpu.VMEM((_NBUF, 1, D), jnp.float32),
            [pltpu.SemaphoreType.DMA] * _NBUF,
            [pltpu.SemaphoreType.DMA] * _NBUF,
        ],
    )
    def gather_k(tt_hbm, idx_hbm, out_hbm, idx_v, gbuf, mini, isems, osems):
        wid = lax.axis_index("s") * NC + lax.axis_index("c")
        base = wid * b_per_w
        pltpu.sync_copy(idx_hbm.at[pl.ds(base, b_per_w)],
                        idx_v.at[pl.ds(0, b_per_w)])
        iota = lax.iota(jnp.int32, 16)

        def fire(u, b):
            grp = pl.multiple_of(
                lax.shift_left(lax.shift_right_logical(u, 7), 7), 128
            )
            pltpu.async_copy(
                tt_hbm.at[:, pl.ds(grp, 128)], gbuf.at[b], isems[b]
            )

        def wait_in(b):
            pltpu.make_async_copy(
                tt_hbm.at[:, pl.ds(0, 128)], gbuf.at[b], isems[b]
            ).wait()

        def wait_out(b):
            pltpu.make_async_copy(
                mini.at[b], out_hbm.at[base], osems[b]
            ).wait()

        def extract_store(vec, j, b, row):
            u = vec[j]
            lane = jnp.full((16,), lax.bitwise_and(u, 127), jnp.int32)
            slot = jnp.full((16,), b, jnp.int32)
            for k in range(D // 16):
                vals = plsc.load_gather(gbuf, [slot, iota + k * 16, lane])
                mini[b, 0, pl.ds(k * 16, 16)] = vals
            pltpu.async_copy(mini.at[b], out_hbm.at[base + row], osems[b])

        vec0 = idx_v[pl.ds(0, 16)]
        for b in range(_NBUF):
            fire(vec0[b], b)

        def body(g, carry):
            vec = idx_v[pl.ds(g * _NBUF, 16)]
            nvec = idx_v[pl.ds((g + 1) * _NBUF, 16)]
            for b in range(_NBUF):
                wait_in(b)

                @pl.when(g > 0)
                def _():
                    wait_out(b)

                extract_store(vec, b, b, g * _NBUF + b)
                fire(nvec[b], b)
            return carry

        lax.fori_loop(0, n_batches - 1, body, 0)

        vec = idx_v[pl.ds((n_batches - 1) * _NBUF, 16)]
        for b in range(_NBUF):
            wait_in(b)
            if n_batches > 1:
                wait_out(b)
            extract_store(vec, b, b, (n_batches - 1) * _NBUF + b)

        if n_rem:
            tvec = idx_v[pl.ds(b_per_w - 16, 16)]
            for t in range(n_rem):
                fire(tvec[16 - n_rem + t], t)
            for t in range(n_rem):
                wait_in(t)
                wait_out(t)
                extract_store(tvec, 16 - n_rem + t, t, b_per_w - n_rem + t)

        for b in range(_NBUF):
            wait_out(b)

    return gather_k


def kernel(userid, table):
    B = userid.shape[0]
    V, D = table.shape
    return _gather_call(B, D, V)(table.T, userid.astype(jnp.int32))


# speedup vs baseline: 1.0262x; 1.0262x over previous
import functools

import jax
import jax.numpy as jnp
from jax import lax
from jax.experimental import pallas as pl
from jax.experimental.pallas import tpu as pltpu
from jax.experimental.pallas import tpu_sc as plsc

_NBUF = 10


def _gather_call(B, D, V):
    info = plsc.get_sparse_core_info()
    NC, NS = info.num_cores, info.num_subcores
    NW = NC * NS
    b_per_w = B // NW
    n_batches = b_per_w // _NBUF
    n_rem = b_per_w % _NBUF

    mesh = plsc.VectorSubcoreMesh(core_axis_name="c", subcore_axis_name="s")

    @functools.partial(
        pl.kernel,
        mesh=mesh,
        out_type=jax.ShapeDtypeStruct((B, 1, D), jnp.float32),
        compiler_params=pltpu.CompilerParams(
            needs_layout_passes=False, disable_bounds_checks=True
        ),
        scratch_types=[
            pltpu.VMEM((b_per_w + 16,), jnp.int32),
            pltpu.VMEM((_NBUF, D, 128), jnp.float32),
            pltpu.VMEM((_NBUF, 1, D), jnp.float32),
            [pltpu.SemaphoreType.DMA] * _NBUF,
            [pltpu.SemaphoreType.DMA] * _NBUF,
        ],
    )
    def gather_k(tt_hbm, idx_hbm, out_hbm, idx_v, gbuf, mini, isems, osems):
        wid = lax.axis_index("s") * NC + lax.axis_index("c")
        base = wid * b_per_w
        pltpu.sync_copy(idx_hbm.at[pl.ds(base, b_per_w)],
                        idx_v.at[pl.ds(0, b_per_w)])
        iota = lax.iota(jnp.int32, 16)

        def fire(u, b):
            grp = pl.multiple_of(
                lax.shift_left(lax.shift_right_logical(u, 7), 7), 128
            )
            pltpu.async_copy(
                tt_hbm.at[:, pl.ds(grp, 128)], gbuf.at[b], isems[b]
            )

        def wait_in(b):
            pltpu.make_async_copy(
                tt_hbm.at[:, pl.ds(0, 128)], gbuf.at[b], isems[b]
            ).wait()

        def wait_out(b):
            pltpu.make_async_copy(
                mini.at[b], out_hbm.at[base], osems[b]
            ).wait()

        def extract_store(vec, j, b, row):
            u = vec[j]
            lane = jnp.full((16,), lax.bitwise_and(u, 127), jnp.int32)
            slot = jnp.full((16,), b, jnp.int32)
            for k in range(D // 16):
                vals = plsc.load_gather(gbuf, [slot, iota + k * 16, lane])
                mini[b, 0, pl.ds(k * 16, 16)] = vals
            pltpu.async_copy(mini.at[b], out_hbm.at[base + row], osems[b])

        vec0 = idx_v[pl.ds(0, 16)]
        for b in range(_NBUF):
            fire(vec0[b], b)

        def body(g, carry):
            vec = idx_v[pl.ds(g * _NBUF, 16)]
            nvec = idx_v[pl.ds((g + 1) * _NBUF, 16)]
            for b in range(_NBUF):
                wait_in(b)

                @pl.when(g > 0)
                def _():
                    wait_out(b)

                extract_store(vec, b, b, g * _NBUF + b)
                fire(nvec[b], b)
            return carry

        lax.fori_loop(0, n_batches - 1, body, 0)

        vec = idx_v[pl.ds((n_batches - 1) * _NBUF, 16)]
        for b in range(_NBUF):
            wait_in(b)
            if n_batches > 1:
                wait_out(b)
            extract_store(vec, b, b, (n_batches - 1) * _NBUF + b)

        if n_rem:
            tvec = idx_v[pl.ds(b_per_w - 16, 16)]
            for t in range(n_rem):
                fire(tvec[16 - n_rem + t], t)
            for t in range(n_rem):
                wait_in(t)
                wait_out(t)
                extract_store(tvec, 16 - n_rem + t, t, b_per_w - n_rem + t)

        for b in range(_NBUF):
            wait_out(b)

    return gather_k


def kernel(userid, table):
    B = userid.shape[0]
    V, D = table.shape
    return _gather_call(B, D, V)(table.T, userid.astype(jnp.int32))
